# baseline (speedup 1.0000x reference)
import functools

import jax
import jax.numpy as jnp
from jax.experimental import pallas as pl
from jax.experimental.pallas import tpu as pltpu

_CDT = jnp.bfloat16
_BBLK = 8


def _unit_body(*refs, stride, Ho, Wo, C1, C2, use1x1, pad_out, fuse_head):
    it = iter(refs)
    x_ref, w1_ref, s1_ref, t1_ref, w2_ref, s2_ref, t2_ref = (
        next(it) for _ in range(7))
    if use1x1:
        ws_ref, tb_ref = next(it), next(it)
    if fuse_head:
        hw1_ref, hb1_ref, hw2_ref, hb2_ref = (next(it) for _ in range(4))
    out_ref = next(it)
    ypad_ref = next(it)

    bblk, Hi, Wi, C = x_ref.shape
    m = bblk * Ho * Wo

    if stride == 2:
        Cc = C // 2
        v5 = x_ref[...].reshape(bblk, Hi // 2, 2, Wi, C)
        parts = [v5[:, dh // 2:dh // 2 + Ho, dh % 2,
                    dw // 2:dw // 2 + Wo, (dw % 2) * Cc:(dw % 2 + 1) * Cc]
                 for dh in range(3) for dw in range(3)]
    else:
        xw = [x_ref[:, :, dw:dw + Wo, :] for dw in range(3)]
        parts = [xw[dw][:, dh:dh + Ho, :, :]
                 for dh in range(3) for dw in range(3)]
    p = jnp.concatenate(parts, axis=-1).reshape(m, -1)
    acc = jnp.dot(p, w1_ref[...], preferred_element_type=jnp.float32)
    y = jnp.maximum(acc * s1_ref[...] + t1_ref[...], 0.0)
    ypad_ref[:, 1:Ho + 1, 1:Wo + 1, :] = y.reshape(
        bblk, Ho, Wo, C1).astype(ypad_ref.dtype)
    zr = jnp.zeros((bblk, 1, Wo + 2, C1), ypad_ref.dtype)
    zc = jnp.zeros((bblk, Ho, 1, C1), ypad_ref.dtype)
    ypad_ref[:, 0:1, :, :] = zr
    ypad_ref[:, Ho + 1:Ho + 2, :, :] = zr
    ypad_ref[:, 1:Ho + 1, 0:1, :] = zc
    ypad_ref[:, 1:Ho + 1, Wo + 1:Wo + 2, :] = zc

    yw = [ypad_ref[:, :, dw:dw + Wo, :] for dw in range(3)]
    parts2 = [yw[dw][:, dh:dh + Ho, :, :]
              for dh in range(3) for dw in range(3)]
    p2 = jnp.concatenate(parts2, axis=-1).reshape(m, 9 * C1)
    acc2 = jnp.dot(p2, w2_ref[...], preferred_element_type=jnp.float32)
    acc2 = acc2 * s2_ref[...] + t2_ref[...]

    if stride == 2:
        xs = v5[:, 0:Ho, 1, 0:Wo, Cc:2 * Cc]
    else:
        xs = x_ref[:, 1:Ho + 1, 1:Wo + 1, :]
    if use1x1:
        sc = jnp.dot(xs.reshape(m, xs.shape[-1]), ws_ref[...],
                     preferred_element_type=jnp.float32) + tb_ref[...]
    else:
        sc = xs.reshape(m, C2).astype(jnp.float32)
    o = jnp.maximum(acc2 + sc, 0.0)

    if fuse_head:
        pooled = jnp.sum(o.reshape(bblk, Ho * Wo, C2), axis=1) * (
            1.0 / float(Ho * Wo))
        h = jnp.dot(pooled, hw1_ref[...],
                    preferred_element_type=jnp.float32) + hb1_ref[...]
        res = (jnp.sum(h * hw2_ref[...], axis=1, keepdims=True)
               + hb2_ref[...])
        out_ref[...] = res.reshape(1, bblk, 1).astype(out_ref.dtype)
    elif pad_out:
        out_ref[:, 1:Ho + 1, 1:Wo + 1, :] = o.reshape(
            bblk, Ho, Wo, C2).astype(out_ref.dtype)
        zr2 = jnp.zeros((bblk, 1, Wo + 2, C2), out_ref.dtype)
        zc2 = jnp.zeros((bblk, Ho, 1, C2), out_ref.dtype)
        out_ref[:, 0:1, :, :] = zr2
        out_ref[:, Ho + 1:Ho + 2, :, :] = zr2
        out_ref[:, 1:Ho + 1, 0:1, :] = zc2
        out_ref[:, 1:Ho + 1, Wo + 1:Wo + 2, :] = zc2
    else:
        out_ref[...] = o.reshape(bblk, Ho, Wo, C2).astype(out_ref.dtype)


def _unit_pp_body(*refs, quad_in, Ho, Wo, C1, C2, use1x1):
    it = iter(refs)
    x_ref, w1_ref, s1_ref, t1_ref, w2_ref, s2_ref, t2_ref = (
        next(it) for _ in range(7))
    if use1x1:
        ws_ref, tb_ref = next(it), next(it)
    out_ref = next(it)
    ypad_ref = next(it)

    x = x_ref[...]
    bblk, Hi, P, L = x.shape
    W2 = Wo // 2
    m2 = bblk * Ho * W2

    def tap4(src, dh, joff, q, cc):
        return src[:, dh:dh + Ho, joff:joff + W2, q * cc:(q + 1) * cc]

    E1, O1 = [(0, 0), (0, 1), (1, 0)], [(0, 1), (1, 0), (1, 1)]

    if quad_in:
        Cin = L // 4
        v5 = x.reshape(bblk, Hi // 2, 2, P, L)

        def tap(dh, joff, q):
            return v5[:, dh // 2:dh // 2 + Ho, dh % 2, joff:joff + W2,
                      q * Cin:(q + 1) * Cin]
        especs, ospecs = [(0, 0), (0, 1), (0, 2)], [(0, 2), (0, 3), (1, 0)]
        pe = [tap(dh, *especs[dw]) for dh in range(3) for dw in range(3)]
        po = [tap(dh, *ospecs[dw]) for dh in range(3) for dw in range(3)]
        xse = v5[:, 0:Ho, 1, 0:W2, Cin:2 * Cin]
        xso = v5[:, 0:Ho, 1, 0:W2, 3 * Cin:4 * Cin]
    else:
        Cin = L // 2
        pe = [tap4(x, dh, *E1[dw], Cin)
              for dh in range(3) for dw in range(3)]
        po = [tap4(x, dh, *O1[dw], Cin)
              for dh in range(3) for dw in range(3)]
        xse = x[:, 1:Ho + 1, 0:W2, Cin:2 * Cin]
        xso = x[:, 1:Ho + 1, 1:1 + W2, 0:Cin]

    def mm_affine(parts, w_ref, s_ref, t_ref):
        pm = jnp.concatenate(parts, axis=-1).reshape(m2, -1)
        a = jnp.dot(pm, w_ref[...], preferred_element_type=jnp.float32)
        return a * s_ref[...] + t_ref[...]

    ye = jnp.maximum(mm_affine(pe, w1_ref, s1_ref, t1_ref), 0.0)
    yo = jnp.maximum(mm_affine(po, w1_ref, s1_ref, t1_ref), 0.0)

    ypad_ref[:, 1:Ho + 1, 1:W2 + 1, 0:C1] = yo.reshape(
        bblk, Ho, W2, C1).astype(ypad_ref.dtype)
    ypad_ref[:, 1:Ho + 1, 0:W2, C1:2 * C1] = ye.reshape(
        bblk, Ho, W2, C1).astype(ypad_ref.dtype)
    zr = jnp.zeros((bblk, 1, W2 + 1, 2 * C1), ypad_ref.dtype)
    ypad_ref[:, 0:1, :, :] = zr
    ypad_ref[:, Ho + 1:Ho + 2, :, :] = zr
    zc = jnp.zeros((bblk, Ho, 1, C1), ypad_ref.dtype)
    ypad_ref[:, 1:Ho + 1, 0:1, 0:C1] = zc
    ypad_ref[:, 1:Ho + 1, W2:W2 + 1, C1:2 * C1] = zc

    yp = ypad_ref[...]
    p2e = [tap4(yp, dh, *E1[dw], C1) for dh in range(3) for dw in range(3)]
    p2o = [tap4(yp, dh, *O1[dw], C1) for dh in range(3) for dw in range(3)]
    acc2e = mm_affine(p2e, w2_ref, s2_ref, t2_ref)
    acc2o = mm_affine(p2o, w2_ref, s2_ref, t2_ref)

    if use1x1:
        sce = jnp.dot(xse.reshape(m2, xse.shape[-1]), ws_ref[...],
                      preferred_element_type=jnp.float32) + tb_ref[...]
        sco = jnp.dot(xso.reshape(m2, xso.shape[-1]), ws_ref[...],
                      preferred_element_type=jnp.float32) + tb_ref[...]
    else:
        sce = xse.reshape(m2, C2).astype(jnp.float32)
        sco = xso.reshape(m2, C2).astype(jnp.float32)
    oe = jnp.maximum(acc2e + sce, 0.0)
    oo = jnp.maximum(acc2o + sco, 0.0)

    out_ref[:, 1:Ho + 1, 1:W2 + 1, 0:C2] = oo.reshape(
        bblk, Ho, W2, C2).astype(out_ref.dtype)
    out_ref[:, 1:Ho + 1, 0:W2, C2:2 * C2] = oe.reshape(
        bblk, Ho, W2, C2).astype(out_ref.dtype)
    zr2 = jnp.zeros((bblk, 1, W2 + 1, 2 * C2), out_ref.dtype)
    out_ref[:, 0:1, :, :] = zr2
    out_ref[:, Ho + 1:Ho + 2, :, :] = zr2
    zc2 = jnp.zeros((bblk, Ho, 1, C2), out_ref.dtype)
    out_ref[:, 1:Ho + 1, 0:1, 0:C2] = zc2
    out_ref[:, 1:Ho + 1, W2:W2 + 1, C2:2 * C2] = zc2


def _pp_unit(x, w1, s1, t1, w2, s2, t2, ws=None, tb=None, *,
             quad_in, Ho, Wo, bblk=4):
    B, Hi, P, L = x.shape
    K1, C1 = w1.shape
    C2 = w2.shape[1]
    use1x1 = ws is not None
    grid = (B // bblk,)
    in_specs = [
        pl.BlockSpec((bblk, Hi, P, L), lambda i: (i, 0, 0, 0)),
        pl.BlockSpec((K1, C1), lambda i: (0, 0)),
        pl.BlockSpec((1, C1), lambda i: (0, 0)),
        pl.BlockSpec((1, C1), lambda i: (0, 0)),
        pl.BlockSpec((9 * C1, C2), lambda i: (0, 0)),
        pl.BlockSpec((1, C2), lambda i: (0, 0)),
        pl.BlockSpec((1, C2), lambda i: (0, 0)),
    ]
    args = [x, w1.astype(_CDT), _row(s1, C1), _row(t1, C1),
            w2.astype(_CDT), _row(s2, C2), _row(t2, C2)]
    if use1x1:
        in_specs += [pl.BlockSpec(ws.shape, lambda i: (0, 0)),
                     pl.BlockSpec((1, C2), lambda i: (0, 0))]
        args += [ws.astype(_CDT), _row(tb, C2)]
    W2 = (Wo + 2) // 2
    out_shape = jax.ShapeDtypeStruct((B, Ho + 2, W2, 2 * C2), _CDT)
    out_spec = pl.BlockSpec((bblk, Ho + 2, W2, 2 * C2),
                            lambda i: (i, 0, 0, 0))
    body = functools.partial(_unit_pp_body, quad_in=quad_in, Ho=Ho, Wo=Wo,
                             C1=C1, C2=C2, use1x1=use1x1)
    return pl.pallas_call(
        body,
        grid=grid,
        in_specs=in_specs,
        out_specs=out_spec,
        out_shape=out_shape,
        scratch_shapes=[pltpu.VMEM((bblk, Ho + 2, W2, 2 * C1), _CDT)],
        compiler_params=pltpu.CompilerParams(
            dimension_semantics=("parallel",)),
    )(*args)


def _fold_bn(conv_bias, gamma, beta, mean, var, eps=1e-5):
    scale = gamma / jnp.sqrt(var + eps)
    shift = (conv_bias - mean) * scale + beta
    return scale, shift


def _row(v, n):
    return v.reshape(1, n).astype(jnp.float32)


def _res_unit(x, w1, s1, t1, w2, s2, t2, ws=None, tb=None, head=None, *,
              stride, Ho, Wo, pad_out, bblk=_BBLK):
    B, Hi, Wi, Ct = x.shape
    x_args = [x]
    x_specs = [pl.BlockSpec((bblk, Hi, Wi, Ct), lambda i: (i, 0, 0, 0))]
    K1, C1 = w1.shape
    C2 = w2.shape[1]
    use1x1 = ws is not None
    fuse_head = head is not None
    grid = (B // bblk,)

    in_specs = x_specs + [
        pl.BlockSpec((K1, C1), lambda i: (0, 0)),
        pl.BlockSpec((1, C1), lambda i: (0, 0)),
        pl.BlockSpec((1, C1), lambda i: (0, 0)),
        pl.BlockSpec((9 * C1, C2), lambda i: (0, 0)),
        pl.BlockSpec((1, C2), lambda i: (0, 0)),
        pl.BlockSpec((1, C2), lambda i: (0, 0)),
    ]
    args = x_args + [w1.astype(_CDT), _row(s1, C1), _row(t1, C1),
                     w2.astype(_CDT), _row(s2, C2), _row(t2, C2)]
    if use1x1:
        in_specs += [pl.BlockSpec(ws.shape, lambda i: (0, 0)),
                     pl.BlockSpec((1, C2), lambda i: (0, 0))]
        args += [ws.astype(_CDT), _row(tb, C2)]
    if fuse_head:
        hw1, hb1, hw2, hb2 = head
        hid = hw1.shape[1]
        in_specs += [pl.BlockSpec((C2, hid), lambda i: (0, 0)),
                     pl.BlockSpec((1, hid), lambda i: (0, 0)),
                     pl.BlockSpec((1, hid), lambda i: (0, 0)),
                     pl.BlockSpec((1, 1), lambda i: (0, 0))]
        args += [hw1.astype(jnp.float32), _row(hb1, hid), _row(hw2, hid),
                 hb2.reshape(1, 1).astype(jnp.float32)]
        out_shape = jax.ShapeDtypeStruct((B // bblk, bblk, 1), jnp.float32)
        out_spec = pl.BlockSpec((1, bblk, 1), lambda i: (i, 0, 0))
    elif pad_out:
        out_shape = jax.ShapeDtypeStruct((B, Ho + 2, Wo + 2, C2), _CDT)
        out_spec = pl.BlockSpec((bblk, Ho + 2, Wo + 2, C2),
                                lambda i: (i, 0, 0, 0))
    else:
        out_shape = jax.ShapeDtypeStruct((B, Ho, Wo, C2), _CDT)
        out_spec = pl.BlockSpec((bblk, Ho, Wo, C2), lambda i: (i, 0, 0, 0))

    body = functools.partial(
        _unit_body, stride=stride, Ho=Ho, Wo=Wo, C1=C1, C2=C2,
        use1x1=use1x1, pad_out=pad_out, fuse_head=fuse_head)
    return pl.pallas_call(
        body,
        grid=grid,
        in_specs=in_specs,
        out_specs=out_spec,
        out_shape=out_shape,
        scratch_shapes=[pltpu.VMEM((bblk, Ho + 2, Wo + 2, C1), _CDT)],
        compiler_params=pltpu.CompilerParams(
            dimension_semantics=("parallel",)),
    )(*args)


def kernel(x, mbh, b0u0_conv1_w, b0u0_conv1_b, b0u0_conv2_w, b0u0_conv2_b, b0u0_bn1_g, b0u0_bn1_be, b0u0_bn1_m, b0u0_bn1_v, b0u0_bn2_g, b0u0_bn2_be, b0u0_bn2_m, b0u0_bn2_v, b0u0_conv3_w, b0u0_conv3_b, b0u1_conv1_w, b0u1_conv1_b, b0u1_conv2_w, b0u1_conv2_b, b0u1_bn1_g, b0u1_bn1_be, b0u1_bn1_m, b0u1_bn1_v, b0u1_bn2_g, b0u1_bn2_be, b0u1_bn2_m, b0u1_bn2_v, b1u0_conv1_w, b1u0_conv1_b, b1u0_conv2_w, b1u0_conv2_b, b1u0_bn1_g, b1u0_bn1_be, b1u0_bn1_m, b1u0_bn1_v, b1u0_bn2_g, b1u0_bn2_be, b1u0_bn2_m, b1u0_bn2_v, b1u0_conv3_w, b1u0_conv3_b, b1u1_conv1_w, b1u1_conv1_b, b1u1_conv2_w, b1u1_conv2_b, b1u1_bn1_g, b1u1_bn1_be, b1u1_bn1_m, b1u1_bn1_v, b1u1_bn2_g, b1u1_bn2_be, b1u1_bn2_m, b1u1_bn2_v, b2u0_conv1_w, b2u0_conv1_b, b2u0_conv2_w, b2u0_conv2_b, b2u0_bn1_g, b2u0_bn1_be, b2u0_bn1_m, b2u0_bn1_v, b2u0_bn2_g, b2u0_bn2_be, b2u0_bn2_m, b2u0_bn2_v, b2u0_conv3_w, b2u0_conv3_b, b2u1_conv1_w, b2u1_conv1_b, b2u1_conv2_w, b2u1_conv2_b, b2u1_bn1_g, b2u1_bn1_be, b2u1_bn1_m, b2u1_bn1_v, b2u1_bn2_g, b2u1_bn2_be, b2u1_bn2_m, b2u1_bn2_v, b3u0_conv1_w, b3u0_conv1_b, b3u0_conv2_w, b3u0_conv2_b, b3u0_bn1_g, b3u0_bn1_be, b3u0_bn1_m, b3u0_bn1_v, b3u0_bn2_g, b3u0_bn2_be, b3u0_bn2_m, b3u0_bn2_v, b3u0_conv3_w, b3u0_conv3_b, b3u1_conv1_w, b3u1_conv1_b, b3u1_conv2_w, b3u1_conv2_b, b3u1_bn1_g, b3u1_bn1_be, b3u1_bn1_m, b3u1_bn1_v, b3u1_bn2_g, b3u1_bn2_be, b3u1_bn2_m, b3u1_bn2_v, w1_feat, w1_mbh, b1, w2, b2):
    xt = jnp.transpose(x, (0, 2, 3, 1)).astype(_CDT)
    x0 = jnp.pad(xt, ((0, 0), (1, 1), (1, 3), (0, 1)))
    x0 = x0.reshape(x0.shape[0], 82, 21, 64)


    def w9(w, cin, cout, cpad=0):
        if cpad:
            w = jnp.pad(w, ((0, 0), (0, 0), (0, cpad), (0, 0)))
        return w.reshape(-1, cout)

    s1, t1 = _fold_bn(b0u0_conv1_b, b0u0_bn1_g, b0u0_bn1_be, b0u0_bn1_m,
                      b0u0_bn1_v)
    s2, t2 = _fold_bn(b0u0_conv2_b, b0u0_bn2_g, b0u0_bn2_be, b0u0_bn2_m,
                      b0u0_bn2_v)
    y = _pp_unit(x0, w9(b0u0_conv1_w, 15, 32, cpad=1), s1, t1,
                 w9(b0u0_conv2_w, 32, 32), s2, t2,
                 ws=jnp.pad(b0u0_conv3_w.reshape(15, 32), ((0, 1), (0, 0))),
                 tb=b0u0_conv3_b, quad_in=True, Ho=40, Wo=40, bblk=8)

    s1, t1 = _fold_bn(b0u1_conv1_b, b0u1_bn1_g, b0u1_bn1_be, b0u1_bn1_m,
                      b0u1_bn1_v)
    s2, t2 = _fold_bn(b0u1_conv2_b, b0u1_bn2_g, b0u1_bn2_be, b0u1_bn2_m,
                      b0u1_bn2_v)
    y = _pp_unit(y, w9(b0u1_conv1_w, 32, 32), s1, t1,
                 w9(b0u1_conv2_w, 32, 32), s2, t2,
                 quad_in=False, Ho=40, Wo=40, bblk=8)

    s1, t1 = _fold_bn(b1u0_conv1_b, b1u0_bn1_g, b1u0_bn1_be, b1u0_bn1_m,
                      b1u0_bn1_v)
    s2, t2 = _fold_bn(b1u0_conv2_b, b1u0_bn2_g, b1u0_bn2_be, b1u0_bn2_m,
                      b1u0_bn2_v)
    y = _res_unit(y, w9(b1u0_conv1_w, 32, 64), s1, t1,
                  w9(b1u0_conv2_w, 64, 64), s2, t2,
                  ws=b1u0_conv3_w.reshape(32, 64), tb=b1u0_conv3_b,
                  stride=2, Ho=20, Wo=20, pad_out=True)

    s1, t1 = _fold_bn(b1u1_conv1_b, b1u1_bn1_g, b1u1_bn1_be, b1u1_bn1_m,
                      b1u1_bn1_v)
    s2, t2 = _fold_bn(b1u1_conv2_b, b1u1_bn2_g, b1u1_bn2_be, b1u1_bn2_m,
                      b1u1_bn2_v)
    y = _res_unit(y, w9(b1u1_conv1_w, 64, 64), s1, t1,
                  w9(b1u1_conv2_w, 64, 64), s2, t2,
                  stride=1, Ho=20, Wo=20, pad_out=True)

    s1, t1 = _fold_bn(b2u0_conv1_b, b2u0_bn1_g, b2u0_bn1_be, b2u0_bn1_m,
                      b2u0_bn1_v)
    s2, t2 = _fold_bn(b2u0_conv2_b, b2u0_bn2_g, b2u0_bn2_be, b2u0_bn2_m,
                      b2u0_bn2_v)
    y = _res_unit(y, w9(b2u0_conv1_w, 64, 128), s1, t1,
                  w9(b2u0_conv2_w, 128, 128), s2, t2,
                  ws=b2u0_conv3_w.reshape(64, 128), tb=b2u0_conv3_b,
                  stride=1, Ho=20, Wo=20, pad_out=True)

    s1, t1 = _fold_bn(b2u1_conv1_b, b2u1_bn1_g, b2u1_bn1_be, b2u1_bn1_m,
                      b2u1_bn1_v)
    s2, t2 = _fold_bn(b2u1_conv2_b, b2u1_bn2_g, b2u1_bn2_be, b2u1_bn2_m,
                      b2u1_bn2_v)
    y = _res_unit(y, w9(b2u1_conv1_w, 128, 128), s1, t1,
                  w9(b2u1_conv2_w, 128, 128), s2, t2,
                  stride=1, Ho=20, Wo=20, pad_out=True)

    s1, t1 = _fold_bn(b3u0_conv1_b, b3u0_bn1_g, b3u0_bn1_be, b3u0_bn1_m,
                      b3u0_bn1_v)
    s2, t2 = _fold_bn(b3u0_conv2_b, b3u0_bn2_g, b3u0_bn2_be, b3u0_bn2_m,
                      b3u0_bn2_v)
    y = _res_unit(y, w9(b3u0_conv1_w, 128, 256), s1, t1,
                  w9(b3u0_conv2_w, 256, 256), s2, t2,
                  ws=b3u0_conv3_w.reshape(128, 256), tb=b3u0_conv3_b,
                  stride=1, Ho=20, Wo=20, pad_out=True, bblk=4)

    s1, t1 = _fold_bn(b3u1_conv1_b, b3u1_bn1_g, b3u1_bn1_be, b3u1_bn1_m,
                      b3u1_bn1_v)
    s2, t2 = _fold_bn(b3u1_conv2_b, b3u1_bn2_g, b3u1_bn2_be, b3u1_bn2_m,
                      b3u1_bn2_v)
    hb1 = b1.reshape(1, -1) + jnp.asarray(mbh, jnp.float32).reshape(1, 1) * \
        w1_mbh.reshape(1, -1)
    out = _res_unit(y, w9(b3u1_conv1_w, 256, 256), s1, t1,
                    w9(b3u1_conv2_w, 256, 256), s2, t2,
                    head=(w1_feat, hb1, w2.reshape(1, -1), b2),
                    stride=1, Ho=20, Wo=20, pad_out=False, bblk=4)
    return out.reshape(-1, 1)


# speedup vs baseline: 1.1156x; 1.1156x over previous
import functools

import jax
import jax.numpy as jnp
from jax.experimental import pallas as pl
from jax.experimental.pallas import tpu as pltpu

_CDT = jnp.bfloat16
_BBLK = 8


def _unit_body(*refs, stride, Ho, Wo, C1, C2, use1x1, pad_out, fuse_head):
    it = iter(refs)
    x_ref, w1_ref, s1_ref, t1_ref, w2_ref, s2_ref, t2_ref = (
        next(it) for _ in range(7))
    if use1x1:
        ws_ref, tb_ref = next(it), next(it)
    if fuse_head:
        hw1_ref, hb1_ref, hw2_ref, hb2_ref = (next(it) for _ in range(4))
    out_ref = next(it)
    ypad_ref = next(it)

    bblk, Hi, Wi, C = x_ref.shape
    m = bblk * Ho * Wo

    if stride == 2:
        Cc = C // 2
        v5 = x_ref[...].astype(_CDT).reshape(bblk, Hi // 2, 2, Wi, C)
        parts = [v5[:, dh // 2:dh // 2 + Ho, dh % 2,
                    dw // 2:dw // 2 + Wo, (dw % 2) * Cc:(dw % 2 + 1) * Cc]
                 for dh in range(3) for dw in range(3)]
    else:
        xw = [x_ref[:, :, dw:dw + Wo, :].astype(_CDT) for dw in range(3)]
        parts = [xw[dw][:, dh:dh + Ho, :, :]
                 for dh in range(3) for dw in range(3)]
    p = jnp.concatenate(parts, axis=-1).reshape(m, -1)
    acc = jnp.dot(p, w1_ref[...], preferred_element_type=jnp.float32)
    y = jnp.maximum(acc * s1_ref[...] + t1_ref[...], 0.0)
    ypad_ref[:, 1:Ho + 1, 1:Wo + 1, :] = y.reshape(
        bblk, Ho, Wo, C1).astype(ypad_ref.dtype)
    zr = jnp.zeros((bblk, 1, Wo + 2, C1), ypad_ref.dtype)
    zc = jnp.zeros((bblk, Ho, 1, C1), ypad_ref.dtype)
    ypad_ref[:, 0:1, :, :] = zr
    ypad_ref[:, Ho + 1:Ho + 2, :, :] = zr
    ypad_ref[:, 1:Ho + 1, 0:1, :] = zc
    ypad_ref[:, 1:Ho + 1, Wo + 1:Wo + 2, :] = zc

    yw = [ypad_ref[:, :, dw:dw + Wo, :] for dw in range(3)]
    parts2 = [yw[dw][:, dh:dh + Ho, :, :]
              for dh in range(3) for dw in range(3)]
    p2 = jnp.concatenate(parts2, axis=-1).reshape(m, 9 * C1)
    acc2 = jnp.dot(p2, w2_ref[...], preferred_element_type=jnp.float32)
    acc2 = acc2 * s2_ref[...] + t2_ref[...]

    if stride == 2:
        xs = v5[:, 0:Ho, 1, 0:Wo, Cc:2 * Cc]
    else:
        xs = x_ref[:, 1:Ho + 1, 1:Wo + 1, :]
    if use1x1:
        sc = jnp.dot(xs.reshape(m, xs.shape[-1]), ws_ref[...],
                     preferred_element_type=jnp.float32) + tb_ref[...]
    else:
        sc = xs.reshape(m, C2).astype(jnp.float32)
    o = jnp.maximum(acc2 + sc, 0.0)

    if fuse_head:
        pooled = jnp.sum(o.reshape(bblk, Ho * Wo, C2), axis=1) * (
            1.0 / float(Ho * Wo))
        h = jnp.dot(pooled, hw1_ref[...],
                    preferred_element_type=jnp.float32) + hb1_ref[...]
        res = (jnp.sum(h * hw2_ref[...], axis=1, keepdims=True)
               + hb2_ref[...])
        out_ref[...] = res.reshape(1, bblk, 1).astype(out_ref.dtype)
    elif pad_out:
        out_ref[:, 1:Ho + 1, 1:Wo + 1, :] = o.reshape(
            bblk, Ho, Wo, C2).astype(out_ref.dtype)
        zr2 = jnp.zeros((bblk, 1, Wo + 2, C2), out_ref.dtype)
        zc2 = jnp.zeros((bblk, Ho, 1, C2), out_ref.dtype)
        out_ref[:, 0:1, :, :] = zr2
        out_ref[:, Ho + 1:Ho + 2, :, :] = zr2
        out_ref[:, 1:Ho + 1, 0:1, :] = zc2
        out_ref[:, 1:Ho + 1, Wo + 1:Wo + 2, :] = zc2
    else:
        out_ref[...] = o.reshape(bblk, Ho, Wo, C2).astype(out_ref.dtype)


def _unit_pp_body(*refs, quad_in, Ho, Wo, C1, C2, use1x1):
    it = iter(refs)
    x_ref, w1_ref, s1_ref, t1_ref, w2_ref, s2_ref, t2_ref = (
        next(it) for _ in range(7))
    if use1x1:
        ws_ref, tb_ref = next(it), next(it)
    out_ref = next(it)
    ypad_ref = next(it)

    x = x_ref[...]
    bblk, Hi, P, L = x.shape
    W2 = Wo // 2
    m2 = bblk * Ho * W2

    def tap4(src, dh, joff, q, cc):
        return src[:, dh:dh + Ho, joff:joff + W2, q * cc:(q + 1) * cc]

    E1, O1 = [(0, 0), (0, 1), (1, 0)], [(0, 1), (1, 0), (1, 1)]

    if quad_in:
        Cin = L // 4
        v5 = x.astype(_CDT).reshape(bblk, Hi // 2, 2, P, L)

        def tap(dh, joff, q):
            return v5[:, dh // 2:dh // 2 + Ho, dh % 2, joff:joff + W2,
                      q * Cin:(q + 1) * Cin]
        especs, ospecs = [(0, 0), (0, 1), (0, 2)], [(0, 2), (0, 3), (1, 0)]
        pe = [tap(dh, *especs[dw]) for dh in range(3) for dw in range(3)]
        po = [tap(dh, *ospecs[dw]) for dh in range(3) for dw in range(3)]
        xse = v5[:, 0:Ho, 1, 0:W2, Cin:2 * Cin]
        xso = v5[:, 0:Ho, 1, 0:W2, 3 * Cin:4 * Cin]
    else:
        Cin = L // 2
        xb = x.astype(_CDT)
        pe = [tap4(xb, dh, *E1[dw], Cin)
              for dh in range(3) for dw in range(3)]
        po = [tap4(xb, dh, *O1[dw], Cin)
              for dh in range(3) for dw in range(3)]
        xse = x[:, 1:Ho + 1, 0:W2, Cin:2 * Cin]
        xso = x[:, 1:Ho + 1, 1:1 + W2, 0:Cin]

    def mm_affine(parts, w_ref, s_ref, t_ref):
        pm = jnp.concatenate(parts, axis=-1).reshape(m2, -1)
        a = jnp.dot(pm, w_ref[...], preferred_element_type=jnp.float32)
        return a * s_ref[...] + t_ref[...]

    ye = jnp.maximum(mm_affine(pe, w1_ref, s1_ref, t1_ref), 0.0)
    yo = jnp.maximum(mm_affine(po, w1_ref, s1_ref, t1_ref), 0.0)

    ypad_ref[:, 1:Ho + 1, 1:W2 + 1, 0:C1] = yo.reshape(
        bblk, Ho, W2, C1).astype(ypad_ref.dtype)
    ypad_ref[:, 1:Ho + 1, 0:W2, C1:2 * C1] = ye.reshape(
        bblk, Ho, W2, C1).astype(ypad_ref.dtype)
    zr = jnp.zeros((bblk, 1, W2 + 1, 2 * C1), ypad_ref.dtype)
    ypad_ref[:, 0:1, :, :] = zr
    ypad_ref[:, Ho + 1:Ho + 2, :, :] = zr
    zc = jnp.zeros((bblk, Ho, 1, C1), ypad_ref.dtype)
    ypad_ref[:, 1:Ho + 1, 0:1, 0:C1] = zc
    ypad_ref[:, 1:Ho + 1, W2:W2 + 1, C1:2 * C1] = zc

    yp = ypad_ref[...]
    p2e = [tap4(yp, dh, *E1[dw], C1) for dh in range(3) for dw in range(3)]
    p2o = [tap4(yp, dh, *O1[dw], C1) for dh in range(3) for dw in range(3)]
    acc2e = mm_affine(p2e, w2_ref, s2_ref, t2_ref)
    acc2o = mm_affine(p2o, w2_ref, s2_ref, t2_ref)

    if use1x1:
        sce = jnp.dot(xse.reshape(m2, xse.shape[-1]), ws_ref[...],
                      preferred_element_type=jnp.float32) + tb_ref[...]
        sco = jnp.dot(xso.reshape(m2, xso.shape[-1]), ws_ref[...],
                      preferred_element_type=jnp.float32) + tb_ref[...]
    else:
        sce = xse.reshape(m2, C2).astype(jnp.float32)
        sco = xso.reshape(m2, C2).astype(jnp.float32)
    oe = jnp.maximum(acc2e + sce, 0.0)
    oo = jnp.maximum(acc2o + sco, 0.0)

    out_ref[:, 1:Ho + 1, 1:W2 + 1, 0:C2] = oo.reshape(
        bblk, Ho, W2, C2).astype(out_ref.dtype)
    out_ref[:, 1:Ho + 1, 0:W2, C2:2 * C2] = oe.reshape(
        bblk, Ho, W2, C2).astype(out_ref.dtype)
    zr2 = jnp.zeros((bblk, 1, W2 + 1, 2 * C2), out_ref.dtype)
    out_ref[:, 0:1, :, :] = zr2
    out_ref[:, Ho + 1:Ho + 2, :, :] = zr2
    zc2 = jnp.zeros((bblk, Ho, 1, C2), out_ref.dtype)
    out_ref[:, 1:Ho + 1, 0:1, 0:C2] = zc2
    out_ref[:, 1:Ho + 1, W2:W2 + 1, C2:2 * C2] = zc2


def _pp_unit(x, w1, s1, t1, w2, s2, t2, ws=None, tb=None, *,
             quad_in, Ho, Wo, bblk=4):
    B, Hi, P, L = x.shape
    K1, C1 = w1.shape
    C2 = w2.shape[1]
    use1x1 = ws is not None
    grid = (B // bblk,)
    in_specs = [
        pl.BlockSpec((bblk, Hi, P, L), lambda i: (i, 0, 0, 0)),
        pl.BlockSpec((K1, C1), lambda i: (0, 0)),
        pl.BlockSpec((1, C1), lambda i: (0, 0)),
        pl.BlockSpec((1, C1), lambda i: (0, 0)),
        pl.BlockSpec((9 * C1, C2), lambda i: (0, 0)),
        pl.BlockSpec((1, C2), lambda i: (0, 0)),
        pl.BlockSpec((1, C2), lambda i: (0, 0)),
    ]
    args = [x, w1.astype(_CDT), _row(s1, C1), _row(t1, C1),
            w2.astype(_CDT), _row(s2, C2), _row(t2, C2)]
    if use1x1:
        in_specs += [pl.BlockSpec(ws.shape, lambda i: (0, 0)),
                     pl.BlockSpec((1, C2), lambda i: (0, 0))]
        args += [ws.astype(_CDT), _row(tb, C2)]
    W2 = (Wo + 2) // 2
    out_shape = jax.ShapeDtypeStruct((B, Ho + 2, W2, 2 * C2), jnp.float32)
    out_spec = pl.BlockSpec((bblk, Ho + 2, W2, 2 * C2),
                            lambda i: (i, 0, 0, 0))
    body = functools.partial(_unit_pp_body, quad_in=quad_in, Ho=Ho, Wo=Wo,
                             C1=C1, C2=C2, use1x1=use1x1)
    return pl.pallas_call(
        body,
        grid=grid,
        in_specs=in_specs,
        out_specs=out_spec,
        out_shape=out_shape,
        scratch_shapes=[pltpu.VMEM((bblk, Ho + 2, W2, 2 * C1), _CDT)],
        compiler_params=pltpu.CompilerParams(
            dimension_semantics=("parallel",)),
    )(*args)


def _fold_bn(conv_bias, gamma, beta, mean, var, eps=1e-5):
    scale = gamma / jnp.sqrt(var + eps)
    shift = (conv_bias - mean) * scale + beta
    return scale, shift


def _row(v, n):
    return v.reshape(1, n).astype(jnp.float32)


def _res_unit(x, w1, s1, t1, w2, s2, t2, ws=None, tb=None, head=None, *,
              stride, Ho, Wo, pad_out, bblk=_BBLK):
    B, Hi, Wi, Ct = x.shape
    x_args = [x]
    x_specs = [pl.BlockSpec((bblk, Hi, Wi, Ct), lambda i: (i, 0, 0, 0))]
    K1, C1 = w1.shape
    C2 = w2.shape[1]
    use1x1 = ws is not None
    fuse_head = head is not None
    grid = (B // bblk,)

    in_specs = x_specs + [
        pl.BlockSpec((K1, C1), lambda i: (0, 0)),
        pl.BlockSpec((1, C1), lambda i: (0, 0)),
        pl.BlockSpec((1, C1), lambda i: (0, 0)),
        pl.BlockSpec((9 * C1, C2), lambda i: (0, 0)),
        pl.BlockSpec((1, C2), lambda i: (0, 0)),
        pl.BlockSpec((1, C2), lambda i: (0, 0)),
    ]
    args = x_args + [w1.astype(_CDT), _row(s1, C1), _row(t1, C1),
                     w2.astype(_CDT), _row(s2, C2), _row(t2, C2)]
    if use1x1:
        in_specs += [pl.BlockSpec(ws.shape, lambda i: (0, 0)),
                     pl.BlockSpec((1, C2), lambda i: (0, 0))]
        args += [ws.astype(_CDT), _row(tb, C2)]
    if fuse_head:
        hw1, hb1, hw2, hb2 = head
        hid = hw1.shape[1]
        in_specs += [pl.BlockSpec((C2, hid), lambda i: (0, 0)),
                     pl.BlockSpec((1, hid), lambda i: (0, 0)),
                     pl.BlockSpec((1, hid), lambda i: (0, 0)),
                     pl.BlockSpec((1, 1), lambda i: (0, 0))]
        args += [hw1.astype(jnp.float32), _row(hb1, hid), _row(hw2, hid),
                 hb2.reshape(1, 1).astype(jnp.float32)]
        out_shape = jax.ShapeDtypeStruct((B // bblk, bblk, 1), jnp.float32)
        out_spec = pl.BlockSpec((1, bblk, 1), lambda i: (i, 0, 0))
    elif pad_out:
        out_shape = jax.ShapeDtypeStruct((B, Ho + 2, Wo + 2, C2),
                                         jnp.float32)
        out_spec = pl.BlockSpec((bblk, Ho + 2, Wo + 2, C2),
                                lambda i: (i, 0, 0, 0))
    else:
        out_shape = jax.ShapeDtypeStruct((B, Ho, Wo, C2), jnp.float32)
        out_spec = pl.BlockSpec((bblk, Ho, Wo, C2), lambda i: (i, 0, 0, 0))

    body = functools.partial(
        _unit_body, stride=stride, Ho=Ho, Wo=Wo, C1=C1, C2=C2,
        use1x1=use1x1, pad_out=pad_out, fuse_head=fuse_head)
    return pl.pallas_call(
        body,
        grid=grid,
        in_specs=in_specs,
        out_specs=out_spec,
        out_shape=out_shape,
        scratch_shapes=[pltpu.VMEM((bblk, Ho + 2, Wo + 2, C1), _CDT)],
        compiler_params=pltpu.CompilerParams(
            dimension_semantics=("parallel",)),
    )(*args)


def kernel(x, mbh, b0u0_conv1_w, b0u0_conv1_b, b0u0_conv2_w, b0u0_conv2_b, b0u0_bn1_g, b0u0_bn1_be, b0u0_bn1_m, b0u0_bn1_v, b0u0_bn2_g, b0u0_bn2_be, b0u0_bn2_m, b0u0_bn2_v, b0u0_conv3_w, b0u0_conv3_b, b0u1_conv1_w, b0u1_conv1_b, b0u1_conv2_w, b0u1_conv2_b, b0u1_bn1_g, b0u1_bn1_be, b0u1_bn1_m, b0u1_bn1_v, b0u1_bn2_g, b0u1_bn2_be, b0u1_bn2_m, b0u1_bn2_v, b1u0_conv1_w, b1u0_conv1_b, b1u0_conv2_w, b1u0_conv2_b, b1u0_bn1_g, b1u0_bn1_be, b1u0_bn1_m, b1u0_bn1_v, b1u0_bn2_g, b1u0_bn2_be, b1u0_bn2_m, b1u0_bn2_v, b1u0_conv3_w, b1u0_conv3_b, b1u1_conv1_w, b1u1_conv1_b, b1u1_conv2_w, b1u1_conv2_b, b1u1_bn1_g, b1u1_bn1_be, b1u1_bn1_m, b1u1_bn1_v, b1u1_bn2_g, b1u1_bn2_be, b1u1_bn2_m, b1u1_bn2_v, b2u0_conv1_w, b2u0_conv1_b, b2u0_conv2_w, b2u0_conv2_b, b2u0_bn1_g, b2u0_bn1_be, b2u0_bn1_m, b2u0_bn1_v, b2u0_bn2_g, b2u0_bn2_be, b2u0_bn2_m, b2u0_bn2_v, b2u0_conv3_w, b2u0_conv3_b, b2u1_conv1_w, b2u1_conv1_b, b2u1_conv2_w, b2u1_conv2_b, b2u1_bn1_g, b2u1_bn1_be, b2u1_bn1_m, b2u1_bn1_v, b2u1_bn2_g, b2u1_bn2_be, b2u1_bn2_m, b2u1_bn2_v, b3u0_conv1_w, b3u0_conv1_b, b3u0_conv2_w, b3u0_conv2_b, b3u0_bn1_g, b3u0_bn1_be, b3u0_bn1_m, b3u0_bn1_v, b3u0_bn2_g, b3u0_bn2_be, b3u0_bn2_m, b3u0_bn2_v, b3u0_conv3_w, b3u0_conv3_b, b3u1_conv1_w, b3u1_conv1_b, b3u1_conv2_w, b3u1_conv2_b, b3u1_bn1_g, b3u1_bn1_be, b3u1_bn1_m, b3u1_bn1_v, b3u1_bn2_g, b3u1_bn2_be, b3u1_bn2_m, b3u1_bn2_v, w1_feat, w1_mbh, b1, w2, b2):
    xt = jnp.transpose(x, (0, 2, 3, 1)).astype(_CDT)
    x0 = jnp.pad(xt, ((0, 0), (1, 1), (1, 3), (0, 1)))
    x0 = x0.reshape(x0.shape[0], 82, 21, 64)


    def w9(w, cin, cout, cpad=0):
        if cpad:
            w = jnp.pad(w, ((0, 0), (0, 0), (0, cpad), (0, 0)))
        return w.reshape(-1, cout)

    s1, t1 = _fold_bn(b0u0_conv1_b, b0u0_bn1_g, b0u0_bn1_be, b0u0_bn1_m,
                      b0u0_bn1_v)
    s2, t2 = _fold_bn(b0u0_conv2_b, b0u0_bn2_g, b0u0_bn2_be, b0u0_bn2_m,
                      b0u0_bn2_v)
    y = _pp_unit(x0, w9(b0u0_conv1_w, 15, 32, cpad=1), s1, t1,
                 w9(b0u0_conv2_w, 32, 32), s2, t2,
                 ws=jnp.pad(b0u0_conv3_w.reshape(15, 32), ((0, 1), (0, 0))),
                 tb=b0u0_conv3_b, quad_in=True, Ho=40, Wo=40, bblk=4)

    s1, t1 = _fold_bn(b0u1_conv1_b, b0u1_bn1_g, b0u1_bn1_be, b0u1_bn1_m,
                      b0u1_bn1_v)
    s2, t2 = _fold_bn(b0u1_conv2_b, b0u1_bn2_g, b0u1_bn2_be, b0u1_bn2_m,
                      b0u1_bn2_v)
    y = _pp_unit(y, w9(b0u1_conv1_w, 32, 32), s1, t1,
                 w9(b0u1_conv2_w, 32, 32), s2, t2,
                 quad_in=False, Ho=40, Wo=40, bblk=4)

    s1, t1 = _fold_bn(b1u0_conv1_b, b1u0_bn1_g, b1u0_bn1_be, b1u0_bn1_m,
                      b1u0_bn1_v)
    s2, t2 = _fold_bn(b1u0_conv2_b, b1u0_bn2_g, b1u0_bn2_be, b1u0_bn2_m,
                      b1u0_bn2_v)
    y = _res_unit(y, w9(b1u0_conv1_w, 32, 64), s1, t1,
                  w9(b1u0_conv2_w, 64, 64), s2, t2,
                  ws=b1u0_conv3_w.reshape(32, 64), tb=b1u0_conv3_b,
                  stride=2, Ho=20, Wo=20, pad_out=True)

    s1, t1 = _fold_bn(b1u1_conv1_b, b1u1_bn1_g, b1u1_bn1_be, b1u1_bn1_m,
                      b1u1_bn1_v)
    s2, t2 = _fold_bn(b1u1_conv2_b, b1u1_bn2_g, b1u1_bn2_be, b1u1_bn2_m,
                      b1u1_bn2_v)
    y = _res_unit(y, w9(b1u1_conv1_w, 64, 64), s1, t1,
                  w9(b1u1_conv2_w, 64, 64), s2, t2,
                  stride=1, Ho=20, Wo=20, pad_out=True)

    s1, t1 = _fold_bn(b2u0_conv1_b, b2u0_bn1_g, b2u0_bn1_be, b2u0_bn1_m,
                      b2u0_bn1_v)
    s2, t2 = _fold_bn(b2u0_conv2_b, b2u0_bn2_g, b2u0_bn2_be, b2u0_bn2_m,
                      b2u0_bn2_v)
    y = _res_unit(y, w9(b2u0_conv1_w, 64, 128), s1, t1,
                  w9(b2u0_conv2_w, 128, 128), s2, t2,
                  ws=b2u0_conv3_w.reshape(64, 128), tb=b2u0_conv3_b,
                  stride=1, Ho=20, Wo=20, pad_out=True)

    s1, t1 = _fold_bn(b2u1_conv1_b, b2u1_bn1_g, b2u1_bn1_be, b2u1_bn1_m,
                      b2u1_bn1_v)
    s2, t2 = _fold_bn(b2u1_conv2_b, b2u1_bn2_g, b2u1_bn2_be, b2u1_bn2_m,
                      b2u1_bn2_v)
    y = _res_unit(y, w9(b2u1_conv1_w, 128, 128), s1, t1,
                  w9(b2u1_conv2_w, 128, 128), s2, t2,
                  stride=1, Ho=20, Wo=20, pad_out=True)

    s1, t1 = _fold_bn(b3u0_conv1_b, b3u0_bn1_g, b3u0_bn1_be, b3u0_bn1_m,
                      b3u0_bn1_v)
    s2, t2 = _fold_bn(b3u0_conv2_b, b3u0_bn2_g, b3u0_bn2_be, b3u0_bn2_m,
                      b3u0_bn2_v)
    y = _res_unit(y, w9(b3u0_conv1_w, 128, 256), s1, t1,
                  w9(b3u0_conv2_w, 256, 256), s2, t2,
                  ws=b3u0_conv3_w.reshape(128, 256), tb=b3u0_conv3_b,
                  stride=1, Ho=20, Wo=20, pad_out=True, bblk=4)

    s1, t1 = _fold_bn(b3u1_conv1_b, b3u1_bn1_g, b3u1_bn1_be, b3u1_bn1_m,
                      b3u1_bn1_v)
    s2, t2 = _fold_bn(b3u1_conv2_b, b3u1_bn2_g, b3u1_bn2_be, b3u1_bn2_m,
                      b3u1_bn2_v)
    hb1 = b1.reshape(1, -1) + jnp.asarray(mbh, jnp.float32).reshape(1, 1) * \
        w1_mbh.reshape(1, -1)
    out = _res_unit(y, w9(b3u1_conv1_w, 256, 256), s1, t1,
                    w9(b3u1_conv2_w, 256, 256), s2, t2,
                    head=(w1_feat, hb1, w2.reshape(1, -1), b2),
                    stride=1, Ho=20, Wo=20, pad_out=False, bblk=4)
    return out.reshape(-1, 1)
